# Initial kernel scaffold; baseline (speedup 1.0000x reference)
import jax
import jax.numpy as jnp
from jax.experimental import pallas as pl


def kernel(x, Wq, Wk, Wv, Wo, W_idx_q, W_idx_k, W_idx_w, b_idx_w, idx_bias, W_vg, b_vg, W_og, b_og):
    raise NotImplementedError("write your pallas kernel here")


# speedup vs baseline: 3.7784x; 3.7784x over previous
import functools
import math

import numpy as np
import jax
import jax.numpy as jnp
from jax import lax
from jax.experimental import pallas as pl
from jax.experimental.pallas import tpu as pltpu

B, T, D = 1, 2048, 768
H = 12
DH = D // H
D_IDX = 64
N_IDX_H = 4
K_SEL = 64

QBLK = 256
NQB = T // QBLK

_SCALE_ATT = DH ** (-0.5)
_SCALE_IDX = 1.0 / math.sqrt(D_IDX)
_NEG = -1e30


def _rope_tables():
    inv_freq = 1.0 / (10000.0 ** (np.arange(0, DH, 2, dtype=np.float32) / np.float32(DH)))
    t = np.arange(T, dtype=np.float32)
    freqs = np.outer(t, inv_freq)
    emb = np.concatenate([freqs, freqs], axis=-1)
    return np.cos(emb).astype(np.float32), np.sin(emb).astype(np.float32)

_COS_NP, _SIN_NP = _rope_tables()


def _proj_body(x_ref, wq_ref, wk_ref, wv_ref, wvg_ref, wog_ref, wiq_ref, wik_ref,
               wiw_ref, biw_ref, bvg_ref, bog_ref, cos_ref, sin_ref,
               q_ref, k_ref, v_ref, og_ref, qi_ref, ki_ref, ws_ref):
    x = x_ref[...]
    f32 = jnp.float32

    def mm(w_ref):
        return jax.lax.dot_general(x, w_ref[...], (((1,), (0,)), ((), ())),
                                   preferred_element_type=f32)

    cos = cos_ref[...]
    sin = sin_ref[...]

    def rope(a):
        a3 = a.reshape(QBLK, H, DH)
        rot = jnp.concatenate([-a3[..., DH // 2:], a3[..., :DH // 2]], axis=-1)
        c = cos.reshape(QBLK, 1, DH)
        s = sin.reshape(QBLK, 1, DH)
        return (a3 * c + rot * s).reshape(QBLK, D)

    q_ref[...] = rope(mm(wq_ref))
    k_ref[...] = rope(mm(wk_ref))
    vgate = jax.nn.sigmoid(mm(wvg_ref) + bvg_ref[...])
    v_ref[...] = mm(wv_ref) * vgate
    og_ref[...] = jax.nn.sigmoid(mm(wog_ref) + bog_ref[...])
    qi_ref[...] = mm(wiq_ref)
    ki_ref[...] = mm(wik_ref)
    ws_ref[...] = jax.nn.sigmoid(mm(wiw_ref) + biw_ref[...])


def _scores_body(qi_ref, ki_ref, ws_ref, bias_ref, out_ref):
    qb = pl.program_id(0)
    kb = pl.program_id(1)

    @pl.when(kb <= qb)
    def _():
        ki = ki_ref[...]
        ws = ws_ref[...]
        acc = jnp.zeros((QBLK, QBLK), jnp.float32)
        for h in range(N_IDX_H):
            qh = qi_ref[:, h * D_IDX:(h + 1) * D_IDX]
            raw = jax.lax.dot_general(qh, ki, (((1,), (1,)), ((), ())),
                                      preferred_element_type=jnp.float32)
            g = jax.nn.sigmoid(raw * _SCALE_IDX + bias_ref[0, h])
            acc = acc + g * ws[:, h:h + 1]
        rows = qb * QBLK + lax.broadcasted_iota(jnp.int32, (QBLK, QBLK), 0)
        cols = kb * QBLK + lax.broadcasted_iota(jnp.int32, (QBLK, QBLK), 1)
        out_ref[...] = jnp.where(cols > rows, -jnp.inf, acc)

    @pl.when(kb > qb)
    def _():
        out_ref[...] = jnp.full((QBLK, QBLK), -jnp.inf, jnp.float32)


def _attn_body(q_ref, k_ref, v_ref, mask_ref, og_ref, wo_ref, out_ref):
    q = q_ref[...]
    mask = mask_ref[...] > 0
    outs = []
    for h in range(H):
        sl = slice(h * DH, (h + 1) * DH)
        qh = q[:, sl]
        kh = k_ref[:, sl]
        s = jax.lax.dot_general(qh, kh, (((1,), (1,)), ((), ())),
                                preferred_element_type=jnp.float32) * _SCALE_ATT
        s = jnp.where(mask, s, _NEG)
        m = jnp.max(s, axis=1, keepdims=True)
        p = jnp.exp(s - m)
        p = jnp.where(mask, p, 0.0)
        denom = jnp.sum(p, axis=1, keepdims=True)
        p = p / denom
        oh = jax.lax.dot_general(p, v_ref[:, sl], (((1,), (0,)), ((), ())),
                                 preferred_element_type=jnp.float32)
        outs.append(oh)
    out = jnp.concatenate(outs, axis=1) * og_ref[...]
    out_ref[...] = jax.lax.dot_general(out, wo_ref[...], (((1,), (0,)), ((), ())),
                                       preferred_element_type=jnp.float32)


def kernel(x, Wq, Wk, Wv, Wo, W_idx_q, W_idx_k, W_idx_w, b_idx_w, idx_bias,
           W_vg, b_vg, W_og, b_og):
    f32 = jnp.float32
    x2 = x.reshape(T, D)
    cos = jnp.asarray(_COS_NP)
    sin = jnp.asarray(_SIN_NP)

    WqT, WkT, WvT = Wq.T, Wk.T, Wv.T
    WvgT, WogT = W_vg.T, W_og.T
    WiqT, WikT = W_idx_q.T, W_idx_k.T
    WiwT = jnp.zeros((D, 128), f32).at[:, :N_IDX_H].set(W_idx_w.T)
    biw = jnp.zeros((1, 128), f32).at[0, :N_IDX_H].set(b_idx_w)
    bias = jnp.zeros((1, 128), f32).at[0, :N_IDX_H].set(idx_bias)
    bvg = b_vg.reshape(1, D)
    bog = b_og.reshape(1, D)

    row_spec = pl.BlockSpec((QBLK, D), lambda i: (i, 0))
    full = lambda shape: pl.BlockSpec(shape, lambda i: (0,) * len(shape))

    q, k, v, og, qi, ki, ws = pl.pallas_call(
        _proj_body,
        grid=(NQB,),
        in_specs=[
            row_spec,
            full((D, D)), full((D, D)), full((D, D)), full((D, D)), full((D, D)),
            full((D, N_IDX_H * D_IDX)), full((D, D_IDX)), full((D, 128)),
            full((1, 128)), full((1, D)), full((1, D)),
            pl.BlockSpec((QBLK, DH), lambda i: (i, 0)),
            pl.BlockSpec((QBLK, DH), lambda i: (i, 0)),
        ],
        out_specs=[
            row_spec, row_spec, row_spec, row_spec,
            pl.BlockSpec((QBLK, N_IDX_H * D_IDX), lambda i: (i, 0)),
            pl.BlockSpec((QBLK, D_IDX), lambda i: (i, 0)),
            pl.BlockSpec((QBLK, 128), lambda i: (i, 0)),
        ],
        out_shape=[
            jax.ShapeDtypeStruct((T, D), f32),
            jax.ShapeDtypeStruct((T, D), f32),
            jax.ShapeDtypeStruct((T, D), f32),
            jax.ShapeDtypeStruct((T, D), f32),
            jax.ShapeDtypeStruct((T, N_IDX_H * D_IDX), f32),
            jax.ShapeDtypeStruct((T, D_IDX), f32),
            jax.ShapeDtypeStruct((T, 128), f32),
        ],
    )(x2, WqT, WkT, WvT, WvgT, WogT, WiqT, WikT, WiwT, biw, bvg, bog, cos, sin)

    scores = pl.pallas_call(
        _scores_body,
        grid=(NQB, NQB),
        in_specs=[
            pl.BlockSpec((QBLK, N_IDX_H * D_IDX), lambda i, j: (i, 0)),
            pl.BlockSpec((QBLK, D_IDX), lambda i, j: (j, 0)),
            pl.BlockSpec((QBLK, 128), lambda i, j: (i, 0)),
            pl.BlockSpec((1, 128), lambda i, j: (0, 0)),
        ],
        out_specs=pl.BlockSpec((QBLK, QBLK), lambda i, j: (i, j)),
        out_shape=jax.ShapeDtypeStruct((T, T), f32),
    )(qi, ki, ws, bias)

    sf = jnp.where(jnp.isneginf(scores), -1e9, scores)
    _, idx = jax.lax.top_k(sf, K_SEL)
    gathered = jnp.take_along_axis(scores, idx, axis=-1)
    valid = jnp.logical_not(jnp.isneginf(gathered))
    rows = jnp.arange(T, dtype=jnp.int32)[:, None]
    mask = jnp.zeros((T, T), f32).at[rows, idx].max(valid.astype(f32))

    out = pl.pallas_call(
        _attn_body,
        grid=(NQB,),
        in_specs=[
            row_spec,
            full((T, D)), full((T, D)),
            pl.BlockSpec((QBLK, T), lambda i: (i, 0)),
            row_spec,
            full((D, D)),
        ],
        out_specs=row_spec,
        out_shape=jax.ShapeDtypeStruct((T, D), f32),
    )(q, k, v, mask, og, Wo.T)

    return out.reshape(B, T, D)
